# baseline (speedup 1.0000x reference)
import functools
import jax
import jax.numpy as jnp
from jax import lax
from jax.experimental import pallas as pl
from jax.experimental.pallas import tpu as pltpu
from jax.experimental.pallas import tpu_sc as plsc

D = 64
NC, NS = 2, 16
NW = NC * NS
BB = 128


def _make_gather(B, L):
    assert B == NW * BB
    n_pairs = L // 2

    @functools.partial(
        pl.kernel,
        mesh=plsc.VectorSubcoreMesh(core_axis_name="c", subcore_axis_name="s"),
        out_type=jax.ShapeDtypeStruct((L, 8, NW, 8, 128), jnp.float32),
        scratch_types=[
            pltpu.VMEM((L, BB), jnp.int32),
            pltpu.VMEM((BB, D), jnp.float32),
            pltpu.VMEM((BB, D), jnp.float32),
            pltpu.VMEM((8, 8, 128), jnp.float32),
            pltpu.VMEM((8, 8, 128), jnp.float32),
            pltpu.SemaphoreType.DMA,
            pltpu.SemaphoreType.DMA,
            pltpu.SemaphoreType.DMA,
            pltpu.SemaphoreType.DMA,
        ],
        compiler_params=pltpu.CompilerParams(
            use_tc_tiling_on_sc=False, needs_layout_passes=False),
    )
    def gather_kernel(idxt_hbm, table_hbm, out_hbm, idx_v,
                      rows0, rows1, stage0, stage1, sg0, sg1, so0, so1):
        rows = [rows0, rows1]
        stage = [stage0, stage1]
        sg = [sg0, sg1]
        so = [so0, so1]
        wid = lax.axis_index("s") * NC + lax.axis_index("c")

        pltpu.sync_copy(idxt_hbm.at[:, pl.ds(wid * BB, BB)], idx_v)

        iota = lax.iota(jnp.int32, 16)
        bvecs = [iota + (16 * b) for b in range(8)]

        def fire_gather(l, p):
            pltpu.async_copy(table_hbm.at[idx_v.at[l]], rows[p], sg[p])

        def drain_gather(p):
            pltpu.make_async_copy(
                table_hbm.at[pl.ds(0, BB)], rows[p], sg[p]
            ).wait()

        def transpose(p):
            def ebody(e, carry):
                te = e // 8
                er = e % 8
                es = jnp.full((16,), e, jnp.int32)
                for b in range(8):
                    v = plsc.load_gather(rows[p], [bvecs[b], es])
                    stage[p][te, er, pl.ds(b * 16, 16)] = v
                return carry

            lax.fori_loop(0, D, ebody, 0)

        def fire_out(l, p):
            pltpu.async_copy(stage[p], out_hbm.at[l, :, wid], so[p])

        def drain_out(p):
            pltpu.make_async_copy(
                stage[p], out_hbm.at[0, :, 0], so[p]
            ).wait()

        fire_gather(0, 0)

        def body(m, carry):
            l = 2 * m
            fire_gather(l + 1, 1)
            drain_gather(0)

            @pl.when(m > 0)
            def _():
                drain_out(0)

            transpose(0)
            fire_out(l, 0)

            @pl.when(m < n_pairs - 1)
            def _():
                fire_gather(l + 2, 0)

            drain_gather(1)

            @pl.when(m > 0)
            def _():
                drain_out(1)

            transpose(1)
            fire_out(l + 1, 1)
            return carry

        lax.fori_loop(0, n_pairs, body, 0)
        drain_out(0)
        drain_out(1)

    return gather_kernel


_gather = _make_gather(4096, 200)


def kernel(batch_data, table):
    idxt = batch_data.T.astype(jnp.int32)
    out5d = _gather(idxt, table)
    out = out5d.transpose((2, 4, 0, 1, 3)).reshape(
        batch_data.shape + (D,))
    return out
